# Initial kernel scaffold; baseline (speedup 1.0000x reference)
import jax
import jax.numpy as jnp
from jax.experimental import pallas as pl


def kernel(ability, labels, wkr_idx, rel_idx, tsk_idx, w_relation, bias):
    raise NotImplementedError("write your pallas kernel here")


# speedup vs baseline: 4.9121x; 4.9121x over previous
import functools

import jax
import jax.numpy as jnp
from jax import lax
from jax.experimental import pallas as pl
from jax.experimental.pallas import tpu as pltpu
from jax.experimental.pallas import tpu_sc as plsc

_LANES = 16
_SUB = 16
_IDXW = 128
_TILE = _SUB * _IDXW
_NW = 32


def _tc_sigmoid_matvec(ability, w_relation, bias):
    n, d = ability.shape
    bm = 10000
    assert n % bm == 0

    def body(a_ref, w_ref, b_ref, p_ref):
        s = jnp.dot(a_ref[...], w_ref[...], preferred_element_type=jnp.float32)
        p_ref[...] = jax.nn.sigmoid(s + b_ref[0, 0])

    return pl.pallas_call(
        body,
        grid=(n // bm,),
        in_specs=[
            pl.BlockSpec((bm, d), lambda i: (i, 0)),
            pl.BlockSpec((d, 1), lambda i: (0, 0)),
            pl.BlockSpec((1, 1), lambda i: (0, 0)),
        ],
        out_specs=pl.BlockSpec((bm, 1), lambda i: (i, 0)),
        out_shape=jax.ShapeDtypeStruct((n, 1), jnp.float32),
    )(ability, w_relation, bias.reshape(1, 1))


def _sc_edge_kernel(num_wkr, num_rels, e_pad):
    steps = e_pad // (_NW * _TILE)
    inv = 1.0 / (num_rels - 1)
    mesh = plsc.VectorSubcoreMesh(core_axis_name="c", subcore_axis_name="s")

    @functools.partial(
        pl.kernel,
        out_type=jax.ShapeDtypeStruct((e_pad,), jnp.float32),
        mesh=mesh,
        compiler_params=pltpu.CompilerParams(needs_layout_passes=False),
        scratch_types=[
            pltpu.VMEM((num_wkr,), jnp.float32),
            pltpu.VMEM((_TILE,), jnp.int32),
            pltpu.VMEM((_TILE,), jnp.int32),
            pltpu.VMEM((_TILE,), jnp.int32),
            pltpu.VMEM((_SUB, _IDXW), jnp.int32),
            pltpu.VMEM((_SUB, _IDXW), jnp.float32),
            pltpu.VMEM((_TILE,), jnp.float32),
            pltpu.SemaphoreType.DMA,
        ],
    )
    def k(p_hbm, lab_hbm, wkr_hbm, tsk_hbm, rel_hbm, out_hbm,
          p_v, wkr_v, tsk_v, rel_v, tidx_v, t_v, out_v, sem):
        wid = lax.axis_index("s") * 2 + lax.axis_index("c")
        pltpu.sync_copy(p_hbm, p_v)

        def step(s, carry):
            off = (wid * steps + s) * _TILE
            pltpu.sync_copy(wkr_hbm.at[pl.ds(off, _TILE)], wkr_v)
            pltpu.sync_copy(tsk_hbm.at[pl.ds(off, _TILE)], tsk_v)
            pltpu.sync_copy(rel_hbm.at[pl.ds(off, _TILE)], rel_v)

            def fuse_idx(r, c2):
                base = r * _IDXW
                for cc in range(_IDXW // _LANES):
                    tk = tsk_v[pl.ds(base + cc * _LANES, _LANES)]
                    rl = rel_v[pl.ds(base + cc * _LANES, _LANES)]
                    tidx_v[r, pl.ds(cc * _LANES, _LANES)] = tk * num_rels + rl
                return c2

            lax.fori_loop(0, _SUB, fuse_idx, 0)

            handles = [
                pltpu.async_copy(lab_hbm.at[tidx_v.at[j]], t_v.at[j], sem)
                for j in range(_SUB)
            ]
            for h in handles:
                h.wait()

            def blend(r, c2):
                base = r * _IDXW
                for cc in range(_IDXW // _LANES):
                    wk = wkr_v[pl.ds(base + cc * _LANES, _LANES)]
                    pv = plsc.load_gather(p_v, [wk])
                    tv = t_v[r, pl.ds(cc * _LANES, _LANES)]
                    out_v[pl.ds(base + cc * _LANES, _LANES)] = (
                        pv * tv + (1.0 - pv) * (1.0 - tv) * inv)
                return c2

            lax.fori_loop(0, _SUB, blend, 0)
            pltpu.sync_copy(out_v, out_hbm.at[pl.ds(off, _TILE)])
            return carry

        lax.fori_loop(0, steps, step, 0)

    return k


def kernel(ability, labels, wkr_idx, rel_idx, tsk_idx, w_relation, bias):
    num_wkr = ability.shape[0]
    num_tsk, _, num_rels = labels.shape
    e = wkr_idx.shape[0]
    chunk = _NW * _TILE
    e_pad = ((e + chunk - 1) // chunk) * chunk

    p = _tc_sigmoid_matvec(ability, w_relation, bias).reshape(num_wkr)
    lab_flat = labels.reshape(num_tsk * num_rels)

    pad = e_pad - e
    wkr_p = jnp.pad(wkr_idx.astype(jnp.int32), (0, pad))
    tsk_p = jnp.pad(tsk_idx.astype(jnp.int32), (0, pad))
    rel_p = jnp.pad(rel_idx.astype(jnp.int32), (0, pad))

    out = _sc_edge_kernel(num_wkr, num_rels, e_pad)(
        p, lab_flat, wkr_p, tsk_p, rel_p)
    return out[:e].reshape(e, 1)


# speedup vs baseline: 4.9343x; 1.0045x over previous
import functools

import jax
import jax.numpy as jnp
from jax import lax
from jax.experimental import pallas as pl
from jax.experimental.pallas import tpu as pltpu
from jax.experimental.pallas import tpu_sc as plsc

_LANES = 16
_SUB = 16
_IDXW = 128
_TILE = _SUB * _IDXW
_NW = 32


def _tc_prologue(ability, w_relation, bias, tsk2d, rel2d, num_rels):
    n, d = ability.shape
    rows = tsk2d.shape[0]
    grid = 4
    bm = n // grid
    br = rows // grid

    def body(a_ref, w_ref, b_ref, t_ref, r_ref, p_ref, x_ref):
        s = jnp.dot(a_ref[...], w_ref[...], preferred_element_type=jnp.float32)
        p_ref[...] = jax.nn.sigmoid(s + b_ref[0, 0])
        x_ref[...] = t_ref[...] * num_rels + r_ref[...]

    return pl.pallas_call(
        body,
        grid=(grid,),
        in_specs=[
            pl.BlockSpec((bm, d), lambda i: (i, 0)),
            pl.BlockSpec((d, 1), lambda i: (0, 0)),
            pl.BlockSpec((1, 1), lambda i: (0, 0)),
            pl.BlockSpec((br, _IDXW), lambda i: (i, 0)),
            pl.BlockSpec((br, _IDXW), lambda i: (i, 0)),
        ],
        out_specs=[
            pl.BlockSpec((bm, 1), lambda i: (i, 0)),
            pl.BlockSpec((br, _IDXW), lambda i: (i, 0)),
        ],
        out_shape=[
            jax.ShapeDtypeStruct((n, 1), jnp.float32),
            jax.ShapeDtypeStruct((rows, _IDXW), jnp.int32),
        ],
    )(ability, w_relation, bias.reshape(1, 1), tsk2d, rel2d)


def _sc_edge_kernel(num_wkr, num_rels, e_pad):
    steps = e_pad // (_NW * _TILE)
    inv = 1.0 / (num_rels - 1)
    mesh = plsc.VectorSubcoreMesh(core_axis_name="c", subcore_axis_name="s")

    @functools.partial(
        pl.kernel,
        out_type=jax.ShapeDtypeStruct((e_pad,), jnp.float32),
        mesh=mesh,
        compiler_params=pltpu.CompilerParams(needs_layout_passes=False),
        scratch_types=[
            pltpu.VMEM((num_wkr,), jnp.float32),
            pltpu.VMEM((2, _TILE), jnp.int32),
            pltpu.VMEM((2, _SUB, _IDXW), jnp.int32),
            pltpu.VMEM((2, _SUB, _IDXW), jnp.float32),
            pltpu.VMEM((_TILE,), jnp.float32),
            pltpu.SemaphoreType.DMA,
            pltpu.SemaphoreType.DMA,
            pltpu.SemaphoreType.DMA,
            pltpu.SemaphoreType.DMA,
            pltpu.SemaphoreType.DMA,
            pltpu.SemaphoreType.DMA,
            pltpu.SemaphoreType.DMA,
        ],
    )
    def k(p_hbm, lab_hbm, wkr_hbm, tidx_hbm, out_hbm,
          p_v, wkr_v, tidx_v, t_v, out_v,
          sem_p, sem_w0, sem_w1, sem_x0, sem_x1, sem_t0, sem_t1):
        sem_w = (sem_w0, sem_w1)
        sem_x = (sem_x0, sem_x1)
        sem_t = (sem_t0, sem_t1)
        wid = lax.axis_index("s") * 2 + lax.axis_index("c")

        def tile_of(s):
            return wid * steps + lax.rem(s, steps)

        def wkr_start(s, b):
            g = tile_of(s)
            pltpu.make_async_copy(
                wkr_hbm.at[pl.ds(g * _TILE, _TILE)], wkr_v.at[b], sem_w[b]
            ).start()

        def wkr_wait(b):
            pltpu.make_async_copy(
                wkr_hbm.at[pl.ds(0, _TILE)], wkr_v.at[b], sem_w[b]).wait()

        def tidx_start(s, b):
            g = tile_of(s)
            pltpu.make_async_copy(
                tidx_hbm.at[pl.ds(g * _SUB, _SUB)], tidx_v.at[b], sem_x[b]
            ).start()

        def tidx_wait(b):
            pltpu.make_async_copy(
                tidx_hbm.at[pl.ds(0, _SUB)], tidx_v.at[b], sem_x[b]).wait()

        def gather_start(b):
            for j in range(_SUB):
                pltpu.make_async_copy(
                    lab_hbm.at[tidx_v.at[b, j]], t_v.at[b, j], sem_t[b]).start()

        def gather_wait(b):
            for j in range(_SUB):
                pltpu.make_async_copy(
                    lab_hbm.at[tidx_v.at[b, j]], t_v.at[b, j], sem_t[b]).wait()

        p_copy = pltpu.make_async_copy(p_hbm, p_v, sem_p)
        p_copy.start()
        tidx_start(0, 0)
        tidx_start(1, 1)
        wkr_start(0, 0)
        wkr_start(1, 1)
        tidx_wait(0)
        gather_start(0)
        p_copy.wait()

        def phase(ph, carry):
            for b in range(2):
                s = 2 * ph + b
                tidx_wait(1 - b)
                gather_start(1 - b)
                gather_wait(b)
                tidx_start(s + 2, b)
                wkr_wait(b)

                def blend(r, c2):
                    base = r * _IDXW
                    for cc in range(_IDXW // _LANES):
                        sl = pl.ds(base + cc * _LANES, _LANES)
                        wk = wkr_v[b, sl]
                        pv = plsc.load_gather(p_v, [wk])
                        tv = t_v[b, r, pl.ds(cc * _LANES, _LANES)]
                        out_v[sl] = pv * tv + (1.0 - pv) * (1.0 - tv) * inv
                    return c2

                lax.fori_loop(0, _SUB, blend, 0)
                pltpu.sync_copy(
                    out_v, out_hbm.at[pl.ds(tile_of(s) * _TILE, _TILE)])
                wkr_start(s + 2, b)
            return carry

        lax.fori_loop(0, steps // 2, phase, 0)

        gather_wait(0)
        tidx_wait(1)
        wkr_wait(0)
        wkr_wait(1)

    return k


def kernel(ability, labels, wkr_idx, rel_idx, tsk_idx, w_relation, bias):
    num_wkr = ability.shape[0]
    num_tsk, _, num_rels = labels.shape
    e = wkr_idx.shape[0]
    chunk = _NW * _TILE
    e_pad = ((e + chunk - 1) // chunk) * chunk

    pad = e_pad - e
    wkr_p = jnp.pad(wkr_idx.astype(jnp.int32), (0, pad))
    tsk2d = jnp.pad(tsk_idx.astype(jnp.int32), (0, pad)).reshape(-1, _IDXW)
    rel2d = jnp.pad(rel_idx.astype(jnp.int32), (0, pad)).reshape(-1, _IDXW)

    p, tidx2d = _tc_prologue(ability, w_relation, bias, tsk2d, rel2d, num_rels)
    lab_flat = labels.reshape(num_tsk * num_rels)

    out = _sc_edge_kernel(num_wkr, num_rels, e_pad)(
        p.reshape(num_wkr), lab_flat, wkr_p, tidx2d)
    return out[:e].reshape(e, 1)


# speedup vs baseline: 5.2892x; 1.0719x over previous
import functools

import jax
import jax.numpy as jnp
from jax import lax
from jax.experimental import pallas as pl
from jax.experimental.pallas import tpu as pltpu
from jax.experimental.pallas import tpu_sc as plsc

_LANES = 16
_SUB = 16
_IDXW = 128
_TILE = _SUB * _IDXW
_NW = 32


def _tc_prologue(ability, w_relation, bias, tsk2d, rel2d, num_rels):
    n, d = ability.shape
    rows = tsk2d.shape[0]
    grid = 4
    bm = 32768
    br = rows // grid

    def body(a_ref, w_ref, b_ref, t_ref, r_ref, p_ref, x_ref):
        s = lax.dot_general(
            w_ref[...], a_ref[...],
            dimension_numbers=(((0,), (1,)), ((), ())),
            preferred_element_type=jnp.float32,
        )
        p_ref[...] = jax.nn.sigmoid(s + b_ref[0, 0])[0]
        x_ref[...] = t_ref[...] * num_rels + r_ref[...]

    return pl.pallas_call(
        body,
        grid=(grid,),
        in_specs=[
            pl.BlockSpec((bm, d), lambda i: (i, 0)),
            pl.BlockSpec((d, 1), lambda i: (0, 0)),
            pl.BlockSpec((1, 1), lambda i: (0, 0)),
            pl.BlockSpec((br, _IDXW), lambda i: (i, 0)),
            pl.BlockSpec((br, _IDXW), lambda i: (i, 0)),
        ],
        out_specs=[
            pl.BlockSpec((bm,), lambda i: (i,)),
            pl.BlockSpec((br, _IDXW), lambda i: (i, 0)),
        ],
        out_shape=[
            jax.ShapeDtypeStruct((n,), jnp.float32),
            jax.ShapeDtypeStruct((rows, _IDXW), jnp.int32),
        ],
    )(ability, w_relation, bias.reshape(1, 1), tsk2d, rel2d)


def _sc_edge_kernel(num_wkr, num_rels, e_pad):
    steps = e_pad // (_NW * _TILE)
    inv = 1.0 / (num_rels - 1)
    mesh = plsc.VectorSubcoreMesh(core_axis_name="c", subcore_axis_name="s")

    @functools.partial(
        pl.kernel,
        out_type=jax.ShapeDtypeStruct((e_pad,), jnp.float32),
        mesh=mesh,
        compiler_params=pltpu.CompilerParams(needs_layout_passes=False),
        scratch_types=[
            pltpu.VMEM((num_wkr,), jnp.float32),
            pltpu.VMEM((2, _TILE), jnp.int32),
            pltpu.VMEM((_TILE,), jnp.int32),
            pltpu.VMEM((_TILE,), jnp.int32),
            pltpu.VMEM((_TILE,), jnp.float32),
            pltpu.VMEM((_TILE,), jnp.float32),
            pltpu.VMEM((_TILE,), jnp.float32),
            pltpu.SemaphoreType.DMA,
            pltpu.SemaphoreType.DMA,
            pltpu.SemaphoreType.DMA,
            pltpu.SemaphoreType.DMA,
            pltpu.SemaphoreType.DMA,
            pltpu.SemaphoreType.DMA,
            pltpu.SemaphoreType.DMA,
        ],
    )
    def k(p_hbm, lab_hbm, wkr_hbm, tidx_hbm, out_hbm,
          p_v, wkr_v, tidx_v0, tidx_v1, t_v0, t_v1, out_v,
          sem_p, sem_w0, sem_w1, sem_x0, sem_x1, sem_t0, sem_t1):
        sem_w = (sem_w0, sem_w1)
        sem_x = (sem_x0, sem_x1)
        sem_t = (sem_t0, sem_t1)
        tidx_v = (tidx_v0, tidx_v1)
        t_v = (t_v0, t_v1)
        wid = lax.axis_index("s") * 2 + lax.axis_index("c")

        def tile_of(s):
            return wid * steps + lax.rem(s, steps)

        def wkr_start(s, b):
            g = tile_of(s)
            pltpu.make_async_copy(
                wkr_hbm.at[pl.ds(g * _TILE, _TILE)], wkr_v.at[b], sem_w[b]
            ).start()

        def wkr_wait(b):
            pltpu.make_async_copy(
                wkr_hbm.at[pl.ds(0, _TILE)], wkr_v.at[b], sem_w[b]).wait()

        def tidx_start(s, b):
            g = tile_of(s)
            pltpu.make_async_copy(
                tidx_hbm.at[pl.ds(g * _TILE, _TILE)], tidx_v[b], sem_x[b]
            ).start()

        def tidx_wait(b):
            pltpu.make_async_copy(
                tidx_hbm.at[pl.ds(0, _TILE)], tidx_v[b], sem_x[b]).wait()

        def gather_start(b):
            pltpu.make_async_copy(
                lab_hbm.at[tidx_v[b]], t_v[b], sem_t[b]).start()

        def gather_wait(b):
            pltpu.make_async_copy(
                lab_hbm.at[tidx_v[b]], t_v[b], sem_t[b]).wait()

        p_copy = pltpu.make_async_copy(p_hbm, p_v, sem_p)
        p_copy.start()
        tidx_start(0, 0)
        tidx_start(1, 1)
        wkr_start(0, 0)
        wkr_start(1, 1)
        tidx_wait(0)
        gather_start(0)
        p_copy.wait()

        def phase(ph, carry):
            for b in range(2):
                s = 2 * ph + b
                tidx_wait(1 - b)
                gather_start(1 - b)
                gather_wait(b)
                tidx_start(s + 2, b)
                wkr_wait(b)

                def blend(r, c2):
                    base = r * _IDXW
                    for cc in range(_IDXW // _LANES):
                        sl = pl.ds(base + cc * _LANES, _LANES)
                        wk = wkr_v[b, sl]
                        pv = plsc.load_gather(p_v, [wk])
                        tv = t_v[b][sl]
                        out_v[sl] = pv * tv + (1.0 - pv) * (1.0 - tv) * inv
                    return c2

                lax.fori_loop(0, _SUB, blend, 0)
                pltpu.sync_copy(
                    out_v, out_hbm.at[pl.ds(tile_of(s) * _TILE, _TILE)])
                wkr_start(s + 2, b)
            return carry

        lax.fori_loop(0, steps // 2, phase, 0)

        gather_wait(0)
        tidx_wait(1)
        wkr_wait(0)
        wkr_wait(1)

    return k


def kernel(ability, labels, wkr_idx, rel_idx, tsk_idx, w_relation, bias):
    num_wkr = ability.shape[0]
    num_tsk, _, num_rels = labels.shape
    e = wkr_idx.shape[0]
    chunk = _NW * _TILE
    e_pad = ((e + chunk - 1) // chunk) * chunk

    pad = e_pad - e
    wkr_p = jnp.pad(wkr_idx.astype(jnp.int32), (0, pad))
    tsk2d = jnp.pad(tsk_idx.astype(jnp.int32), (0, pad)).reshape(-1, _IDXW)
    rel2d = jnp.pad(rel_idx.astype(jnp.int32), (0, pad)).reshape(-1, _IDXW)

    p, tidx2d = _tc_prologue(ability, w_relation, bias, tsk2d, rel2d, num_rels)
    lab_flat = labels.reshape(num_tsk * num_rels)

    out = _sc_edge_kernel(num_wkr, num_rels, e_pad)(
        p, lab_flat, wkr_p, tidx2d.reshape(e_pad))
    return out[:e].reshape(e, 1)
